# Initial kernel scaffold; baseline (speedup 1.0000x reference)
import jax
import jax.numpy as jnp
from jax import lax
from jax.experimental import pallas as pl
from jax.experimental.pallas import tpu as pltpu
from jax.experimental.pallas import tpu_sc as plsc

N = 10000
E = 320000
F_IN = 128
H = 8
C1 = 20
D1 = H * C1
D2 = 16

NP = 10240
D1P = D1 + 16
D2P = 32
NCORE = 2
NSUB = 16
NW = NCORE * NSUB
CH = 128
CPT = 81
PT = CH * CPT
EP = NW * PT
RPT = NP // NSUB
BR = 512


def _mesh():
    return plsc.VectorSubcoreMesh(
        core_axis_name="c", subcore_axis_name="s",
        num_cores=NCORE, num_subcores=NSUB)


def _tca_body(x_ref, w1_ref, as_ref, ad_ref, h_ref, ts_ref, td_ref):
    h = jnp.dot(x_ref[...], w1_ref[...], preferred_element_type=jnp.float32)
    h_ref[...] = h
    ts_ref[...] = jnp.dot(h, as_ref[...], preferred_element_type=jnp.float32)
    td_ref[...] = jnp.dot(h, ad_ref[...], preferred_element_type=jnp.float32)


def _tca(x_pad, W1, As16, Ad16):
    return pl.pallas_call(
        _tca_body,
        grid=(NP // BR,),
        in_specs=[
            pl.BlockSpec((BR, F_IN), lambda i: (i, 0)),
            pl.BlockSpec((F_IN, D1), lambda i: (0, 0)),
            pl.BlockSpec((D1, 16), lambda i: (0, 0)),
            pl.BlockSpec((D1, 16), lambda i: (0, 0)),
        ],
        out_specs=[
            pl.BlockSpec((BR, D1), lambda i: (i, 0)),
            pl.BlockSpec((BR, 16), lambda i: (i, 0)),
            pl.BlockSpec((BR, 16), lambda i: (i, 0)),
        ],
        out_shape=[
            jax.ShapeDtypeStruct((NP, D1), jnp.float32),
            jax.ShapeDtypeStruct((NP, 16), jnp.float32),
            jax.ShapeDtypeStruct((NP, 16), jnp.float32),
        ],
    )(x_pad, W1, As16, Ad16)


def _sc1_body(src_hbm, dst_hbm, ts_hbm, td_hbm, h_hbm, out_hbm,
              src_v, dst_v, as_v, ad_v, w_v, h_v, msg_v, acc_sh,
              sem0, sem1, sem2):
    cid = lax.axis_index("c")
    sid = lax.axis_index("s")
    wid = sid * NCORE + cid

    iota = lax.iota(jnp.int32, 16)
    hmask = jnp.where(iota < H, 1.0, 0.0).astype(jnp.float32)
    hmaps = [(iota + 16 * k) // C1 for k in range(D1 // 16)]
    zz = jnp.zeros((16,), jnp.float32)

    @pl.loop(0, CH)
    def _(i):
        for k in range(D1P // 16):
            msg_v[i, pl.ds(k * 16, 16)] = zz

    row0 = sid * RPT
    for j in range(RPT // CH):
        pltpu.sync_copy(msg_v, acc_sh.at[pl.ds(row0 + j * CH, CH)])
    plsc.subcore_barrier()

    @pl.loop(0, CPT)
    def _(ci):
        ebase = wid * PT + ci * CH
        pltpu.sync_copy(src_hbm.at[pl.ds(ebase, CH)], src_v)
        pltpu.sync_copy(dst_hbm.at[pl.ds(ebase, CH)], dst_v)
        c1 = pltpu.async_copy(ts_hbm.at[src_v], as_v, sem0)
        c2 = pltpu.async_copy(td_hbm.at[dst_v], ad_v, sem1)
        c3 = pltpu.async_copy(h_hbm.at[src_v], h_v, sem2)
        c1.wait()
        c2.wait()

        @pl.loop(0, CH)
        def _(i):
            s = as_v[i] + ad_v[i]
            w_v[pl.ds(i * 16, 16)] = jnp.exp(jnp.maximum(s, 0.2 * s))

        c3.wait()

        @pl.loop(0, CH)
        def _(i):
            b16 = i * 16
            wrow = w_v[pl.ds(b16, 16)]
            msg_v[i, pl.ds(D1, 16)] = wrow * hmask
            for k in range(D1 // 16):
                sc = plsc.load_gather(w_v, [b16 + hmaps[k]])
                msg_v[i, pl.ds(k * 16, 16)] = h_v[i, pl.ds(k * 16, 16)] * sc

        pltpu.sync_copy(msg_v, acc_sh.at[dst_v], add=True)

    plsc.subcore_barrier()
    ob = cid * NP + row0
    for j in range(RPT // CH):
        pltpu.sync_copy(acc_sh.at[pl.ds(row0 + j * CH, CH)], msg_v)
        pltpu.sync_copy(msg_v, out_hbm.at[pl.ds(ob + j * CH, CH)])


def _sc1(srcE, dstE, ts, td, h1):
    k = pl.kernel(
        _sc1_body,
        out_type=jax.ShapeDtypeStruct((NCORE * NP, D1P), jnp.float32),
        mesh=_mesh(),
        scratch_types=[
            pltpu.VMEM((CH,), jnp.int32),
            pltpu.VMEM((CH,), jnp.int32),
            pltpu.VMEM((CH, 16), jnp.float32),
            pltpu.VMEM((CH, 16), jnp.float32),
            pltpu.VMEM((CH * 16,), jnp.float32),
            pltpu.VMEM((CH, D1), jnp.float32),
            pltpu.VMEM((CH, D1P), jnp.float32),
            pltpu.VMEM_SHARED((NP, D1P), jnp.float32),
            pltpu.SemaphoreType.DMA,
            pltpu.SemaphoreType.DMA,
            pltpu.SemaphoreType.DMA,
        ],
    )
    return k(srcE, dstE, ts, td, h1)


def _tcb_body(a_ref, b_ref, w2_ref, b1_ref, rexp_ref, as2_ref, ad2_ref,
              h2_ref, s2_ref, d2_ref):
    acc = a_ref[...] + b_ref[...]
    num = acc[:, :D1]
    den = acc[:, D1:D1 + H]
    rec = 1.0 / (den + 1e-16)
    rec160 = jnp.dot(rec, rexp_ref[...], preferred_element_type=jnp.float32)
    out1 = jnp.maximum(num * rec160 + b1_ref[...], 0.0)
    h2 = jnp.dot(out1, w2_ref[...], preferred_element_type=jnp.float32)
    h2_ref[...] = h2
    s2_ref[...] = jnp.sum(h2 * as2_ref[...], axis=1)
    d2_ref[...] = jnp.sum(h2 * ad2_ref[...], axis=1)


def _tcb(acc1, W2, b1r, Rexp, as2r, ad2r):
    return pl.pallas_call(
        _tcb_body,
        grid=(NP // BR,),
        in_specs=[
            pl.BlockSpec((BR, D1P), lambda i: (i, 0)),
            pl.BlockSpec((BR, D1P), lambda i: (i + NP // BR, 0)),
            pl.BlockSpec((D1, D2), lambda i: (0, 0)),
            pl.BlockSpec((1, D1), lambda i: (0, 0)),
            pl.BlockSpec((H, D1), lambda i: (0, 0)),
            pl.BlockSpec((1, D2), lambda i: (0, 0)),
            pl.BlockSpec((1, D2), lambda i: (0, 0)),
        ],
        out_specs=[
            pl.BlockSpec((BR, D2), lambda i: (i, 0)),
            pl.BlockSpec((BR,), lambda i: (i,)),
            pl.BlockSpec((BR,), lambda i: (i,)),
        ],
        out_shape=[
            jax.ShapeDtypeStruct((NP, D2), jnp.float32),
            jax.ShapeDtypeStruct((NP,), jnp.float32),
            jax.ShapeDtypeStruct((NP,), jnp.float32),
        ],
    )(acc1, acc1, W2, b1r, Rexp, as2r, ad2r)


def _sc2_body(src_hbm, dst_hbm, as2_hbm, ad2_hbm, h2_hbm, out_hbm,
              src_v, dst_v, as2_v, ad2_v, w_v, h2_v, msg_v, acc_sh, sem0):
    cid = lax.axis_index("c")
    sid = lax.axis_index("s")
    wid = sid * NCORE + cid

    iota = lax.iota(jnp.int32, 16)
    l0mask = jnp.where(iota == 0, 1.0, 0.0).astype(jnp.float32)
    zz = jnp.zeros((16,), jnp.float32)

    pltpu.sync_copy(as2_hbm, as2_v)
    pltpu.sync_copy(ad2_hbm, ad2_v)

    @pl.loop(0, CH)
    def _(i):
        msg_v[i, pl.ds(0, 16)] = zz
        msg_v[i, pl.ds(16, 16)] = zz

    row0 = sid * RPT
    for j in range(RPT // CH):
        pltpu.sync_copy(msg_v, acc_sh.at[pl.ds(row0 + j * CH, CH)])
    plsc.subcore_barrier()

    @pl.loop(0, CPT)
    def _(ci):
        ebase = wid * PT + ci * CH
        pltpu.sync_copy(src_hbm.at[pl.ds(ebase, CH)], src_v)
        pltpu.sync_copy(dst_hbm.at[pl.ds(ebase, CH)], dst_v)
        c3 = pltpu.async_copy(h2_hbm.at[src_v], h2_v, sem0)

        @pl.loop(0, CH // 16)
        def _(j):
            sv = src_v[pl.ds(j * 16, 16)]
            dv = dst_v[pl.ds(j * 16, 16)]
            s = plsc.load_gather(as2_v, [sv]) + plsc.load_gather(ad2_v, [dv])
            w_v[pl.ds(j * 16, 16)] = jnp.exp(jnp.maximum(s, 0.2 * s))

        c3.wait()

        @pl.loop(0, CH)
        def _(i):
            sc = plsc.load_gather(w_v, [iota * 0 + i])
            msg_v[i, pl.ds(0, 16)] = h2_v[i] * sc
            msg_v[i, pl.ds(16, 16)] = sc * l0mask

        pltpu.sync_copy(msg_v, acc_sh.at[dst_v], add=True)

    plsc.subcore_barrier()
    ob = cid * NP + row0
    for j in range(RPT // CH):
        pltpu.sync_copy(acc_sh.at[pl.ds(row0 + j * CH, CH)], msg_v)
        pltpu.sync_copy(msg_v, out_hbm.at[pl.ds(ob + j * CH, CH)])


def _sc2(srcE, dstE, as2, ad2, h2):
    k = pl.kernel(
        _sc2_body,
        out_type=jax.ShapeDtypeStruct((NCORE * NP, D2P), jnp.float32),
        mesh=_mesh(),
        scratch_types=[
            pltpu.VMEM((CH,), jnp.int32),
            pltpu.VMEM((CH,), jnp.int32),
            pltpu.VMEM((NP,), jnp.float32),
            pltpu.VMEM((NP,), jnp.float32),
            pltpu.VMEM((CH,), jnp.float32),
            pltpu.VMEM((CH, D2), jnp.float32),
            pltpu.VMEM((CH, D2P), jnp.float32),
            pltpu.VMEM_SHARED((NP, D2P), jnp.float32),
            pltpu.SemaphoreType.DMA,
        ],
    )
    return k(srcE, dstE, as2, ad2, h2)


def _tcc_body(a_ref, b_ref, b2_ref, o_ref):
    acc = a_ref[...] + b_ref[...]
    den = acc[:, D2:D2 + 1]
    o_ref[...] = acc[:, :D2] / (den + 1e-16) + b2_ref[...]


def _tcc(acc2, b2r):
    return pl.pallas_call(
        _tcc_body,
        grid=(NP // BR,),
        in_specs=[
            pl.BlockSpec((BR, D2P), lambda i: (i, 0)),
            pl.BlockSpec((BR, D2P), lambda i: (i + NP // BR, 0)),
            pl.BlockSpec((1, D2), lambda i: (0, 0)),
        ],
        out_specs=pl.BlockSpec((BR, D2), lambda i: (i, 0)),
        out_shape=jax.ShapeDtypeStruct((NP, D2), jnp.float32),
    )(acc2, acc2, b2r)


def kernel(x, edge_index, W1, a_src1, a_dst1, b1, W2, a_src2, a_dst2, b2):
    loop = jnp.arange(N, dtype=jnp.int32)
    padi = jnp.full((EP - E - N,), N, dtype=jnp.int32)
    srcE = jnp.concatenate([edge_index[0], loop, padi])
    dstE = jnp.concatenate([edge_index[1], loop, padi])

    x_pad = jnp.zeros((NP, F_IN), jnp.float32).at[:N].set(x)

    idx = jnp.arange(D1)
    As16 = jnp.zeros((D1, 16), jnp.float32).at[idx, idx // C1].set(
        a_src1.reshape(-1))
    Ad16 = jnp.zeros((D1, 16), jnp.float32).at[idx, idx // C1].set(
        a_dst1.reshape(-1))
    Rexp = jnp.zeros((H, D1), jnp.float32).at[idx // C1, idx].set(1.0)

    b1r = b1.reshape(1, D1)
    b2r = b2.reshape(1, D2)

    h1, ts, td = _tca(x_pad, W1, As16, Ad16)
    acc1 = _sc1(srcE, dstE, ts, td, h1)
    h2, as2, ad2 = _tcb(acc1, W2, b1r, Rexp, a_src2, a_dst2)
    acc2 = _sc2(srcE, dstE, as2, ad2, h2)
    out = _tcc(acc2, b2r)
    return out[:N]


# speedup vs baseline: 33.4123x; 33.4123x over previous
import jax
import jax.numpy as jnp
from jax import lax
from jax.experimental import pallas as pl
from jax.experimental.pallas import tpu as pltpu
from jax.experimental.pallas import tpu_sc as plsc

N = 10000
E = 320000
F_IN = 128
H = 8
C1 = 20
D1 = H * C1
DH = D1 // 2
D2 = 16

NP = 10240
DHP = DH + 16
D2P = 32
NCORE = 2
NSUB = 16
NW = NCORE * NSUB
CH = 128
CPT = 81
PT = CH * CPT
EP = NW * PT
RPT = NP // NSUB
BR = 512


def _mesh():
    return plsc.VectorSubcoreMesh(
        core_axis_name="c", subcore_axis_name="s",
        num_cores=NCORE, num_subcores=NSUB)


_SC_PARAMS = pltpu.CompilerParams(
    needs_layout_passes=False, use_tc_tiling_on_sc=False)


def _tca_body(x_ref, wa_ref, wb_ref, asa_ref, asb_ref, ada_ref, adb_ref,
              ha_ref, hb_ref, ts_ref, td_ref):
    x = x_ref[...]
    ha = jnp.dot(x, wa_ref[...], preferred_element_type=jnp.float32)
    hb = jnp.dot(x, wb_ref[...], preferred_element_type=jnp.float32)
    ha_ref[...] = ha
    hb_ref[...] = hb
    ts_ref[...] = (
        jnp.dot(ha, asa_ref[...], preferred_element_type=jnp.float32)
        + jnp.dot(hb, asb_ref[...], preferred_element_type=jnp.float32))
    td_ref[...] = (
        jnp.dot(ha, ada_ref[...], preferred_element_type=jnp.float32)
        + jnp.dot(hb, adb_ref[...], preferred_element_type=jnp.float32))


def _tca(x_pad, W1a, W1b, AsA, AsB, AdA, AdB):
    full = lambda r, c: pl.BlockSpec((r, c), lambda i: (0, 0))
    return pl.pallas_call(
        _tca_body,
        grid=(NP // BR,),
        in_specs=[
            pl.BlockSpec((BR, F_IN), lambda i: (i, 0)),
            full(F_IN, DH), full(F_IN, DH),
            full(DH, 16), full(DH, 16), full(DH, 16), full(DH, 16),
        ],
        out_specs=[
            pl.BlockSpec((BR, DH), lambda i: (i, 0)),
            pl.BlockSpec((BR, DH), lambda i: (i, 0)),
            pl.BlockSpec((BR, 16), lambda i: (i, 0)),
            pl.BlockSpec((BR, 16), lambda i: (i, 0)),
        ],
        out_shape=[
            jax.ShapeDtypeStruct((NP, DH), jnp.float32),
            jax.ShapeDtypeStruct((NP, DH), jnp.float32),
            jax.ShapeDtypeStruct((NP, 16), jnp.float32),
            jax.ShapeDtypeStruct((NP, 16), jnp.float32),
        ],
    )(x_pad, W1a, W1b, AsA, AsB, AdA, AdB)


def _sc1_body(src_hbm, dst_hbm, ts_hbm, td_hbm, ha_hbm, hb_hbm, out_hbm,
              src_v, dst_v, as_v, ad_v, w_v, h_v, msg_v, acc_sh,
              sem0, sem1, sem2):
    cid = lax.axis_index("c")
    sid = lax.axis_index("s")
    wid = sid * NCORE + cid

    iota = lax.iota(jnp.int32, 16)
    hmask = jnp.where(iota < H, 1.0, 0.0).astype(jnp.float32)
    zz = jnp.zeros((16,), jnp.float32)
    row0 = sid * RPT

    for phase, h_hbm in ((0, ha_hbm), (1, hb_hbm)):
        hmaps = [(iota + 16 * k) // C1 + 4 * phase for k in range(DH // 16)]

        @pl.loop(0, CH)
        def _(i):
            for k in range(DHP // 16):
                msg_v[i, pl.ds(k * 16, 16)] = zz

        for j in range(RPT // CH):
            pltpu.sync_copy(msg_v, acc_sh.at[pl.ds(row0 + j * CH, CH)])
        plsc.subcore_barrier()

        @pl.loop(0, CPT)
        def _(ci):
            ebase = wid * PT + ci * CH
            pltpu.sync_copy(src_hbm.at[pl.ds(ebase, CH)], src_v)
            pltpu.sync_copy(dst_hbm.at[pl.ds(ebase, CH)], dst_v)
            c1 = pltpu.async_copy(ts_hbm.at[src_v], as_v, sem0)
            c2 = pltpu.async_copy(td_hbm.at[dst_v], ad_v, sem1)
            c3 = pltpu.async_copy(h_hbm.at[src_v], h_v, sem2)
            c1.wait()
            c2.wait()

            @pl.loop(0, CH)
            def _(i):
                s = as_v[i] + ad_v[i]
                w_v[pl.ds(i * 16, 16)] = jnp.exp(jnp.maximum(s, 0.2 * s))

            c3.wait()

            @pl.loop(0, CH)
            def _(i):
                b16 = i * 16
                wrow = w_v[pl.ds(b16, 16)]
                msg_v[i, pl.ds(DH, 16)] = wrow * hmask
                for k in range(DH // 16):
                    sc = plsc.load_gather(w_v, [b16 + hmaps[k]])
                    msg_v[i, pl.ds(k * 16, 16)] = (
                        h_v[i, pl.ds(k * 16, 16)] * sc)

            pltpu.sync_copy(msg_v, acc_sh.at[dst_v], add=True)

        plsc.subcore_barrier()
        ob = (cid * 2 + phase) * NP + row0
        for j in range(RPT // CH):
            pltpu.sync_copy(acc_sh.at[pl.ds(row0 + j * CH, CH)], msg_v)
            pltpu.sync_copy(msg_v, out_hbm.at[pl.ds(ob + j * CH, CH)])
        plsc.subcore_barrier()


def _sc1(srcE, dstE, ts, td, ha, hb):
    k = pl.kernel(
        _sc1_body,
        out_type=jax.ShapeDtypeStruct((NCORE * 2 * NP, DHP), jnp.float32),
        mesh=_mesh(),
        compiler_params=_SC_PARAMS,
        scratch_types=[
            pltpu.VMEM((CH,), jnp.int32),
            pltpu.VMEM((CH,), jnp.int32),
            pltpu.VMEM((CH, 16), jnp.float32),
            pltpu.VMEM((CH, 16), jnp.float32),
            pltpu.VMEM((CH * 16,), jnp.float32),
            pltpu.VMEM((CH, DH), jnp.float32),
            pltpu.VMEM((CH, DHP), jnp.float32),
            pltpu.VMEM_SHARED((NP, DHP), jnp.float32),
            pltpu.SemaphoreType.DMA,
            pltpu.SemaphoreType.DMA,
            pltpu.SemaphoreType.DMA,
        ],
    )
    return k(srcE, dstE, ts, td, ha, hb)


def _tcb_body(p0a_ref, p0b_ref, p1a_ref, p1b_ref, w2a_ref, w2b_ref,
              b1_ref, rexp_ref, as2_ref, ad2_ref,
              h2_ref, s2_ref, d2_ref):
    accA = p0a_ref[...] + p0b_ref[...]
    accB = p1a_ref[...] + p1b_ref[...]
    den = accA[:, DH:DH + H]
    rec = 1.0 / (den + 1e-16)
    rec160 = jnp.dot(rec, rexp_ref[...], preferred_element_type=jnp.float32)
    outA = jnp.maximum(
        accA[:, :DH] * rec160[:, :DH] + b1_ref[:, :DH], 0.0)
    outB = jnp.maximum(
        accB[:, :DH] * rec160[:, DH:] + b1_ref[:, DH:], 0.0)
    h2 = (jnp.dot(outA, w2a_ref[...], preferred_element_type=jnp.float32)
          + jnp.dot(outB, w2b_ref[...], preferred_element_type=jnp.float32))
    h2_ref[...] = h2
    s2_ref[...] = jnp.sum(h2 * as2_ref[...], axis=1)
    d2_ref[...] = jnp.sum(h2 * ad2_ref[...], axis=1)


def _tcb(acc1, W2a, W2b, b1r, Rexp, as2r, ad2r):
    full = lambda r, c: pl.BlockSpec((r, c), lambda i: (0, 0))
    nb = NP // BR
    return pl.pallas_call(
        _tcb_body,
        grid=(nb,),
        in_specs=[
            pl.BlockSpec((BR, DHP), lambda i: (i, 0)),
            pl.BlockSpec((BR, DHP), lambda i: (i + 2 * nb, 0)),
            pl.BlockSpec((BR, DHP), lambda i: (i + nb, 0)),
            pl.BlockSpec((BR, DHP), lambda i: (i + 3 * nb, 0)),
            full(DH, D2), full(DH, D2),
            full(1, D1), full(H, D1), full(1, D2), full(1, D2),
        ],
        out_specs=[
            pl.BlockSpec((BR, D2), lambda i: (i, 0)),
            pl.BlockSpec((BR,), lambda i: (i,)),
            pl.BlockSpec((BR,), lambda i: (i,)),
        ],
        out_shape=[
            jax.ShapeDtypeStruct((NP, D2), jnp.float32),
            jax.ShapeDtypeStruct((NP,), jnp.float32),
            jax.ShapeDtypeStruct((NP,), jnp.float32),
        ],
    )(acc1, acc1, acc1, acc1, W2a, W2b, b1r, Rexp, as2r, ad2r)


def _sc2_body(src_hbm, dst_hbm, as2_hbm, ad2_hbm, h2_hbm, out_hbm,
              src_v, dst_v, as2_v, ad2_v, w_v, h2_v, msg_v, acc_sh, sem0):
    cid = lax.axis_index("c")
    sid = lax.axis_index("s")
    wid = sid * NCORE + cid

    iota = lax.iota(jnp.int32, 16)
    l0mask = jnp.where(iota == 0, 1.0, 0.0).astype(jnp.float32)
    zz = jnp.zeros((16,), jnp.float32)

    pltpu.sync_copy(as2_hbm, as2_v)
    pltpu.sync_copy(ad2_hbm, ad2_v)

    @pl.loop(0, CH)
    def _(i):
        msg_v[i, pl.ds(0, 16)] = zz
        msg_v[i, pl.ds(16, 16)] = zz

    row0 = sid * RPT
    for j in range(RPT // CH):
        pltpu.sync_copy(msg_v, acc_sh.at[pl.ds(row0 + j * CH, CH)])
    plsc.subcore_barrier()

    @pl.loop(0, CPT)
    def _(ci):
        ebase = wid * PT + ci * CH
        pltpu.sync_copy(src_hbm.at[pl.ds(ebase, CH)], src_v)
        pltpu.sync_copy(dst_hbm.at[pl.ds(ebase, CH)], dst_v)
        c3 = pltpu.async_copy(h2_hbm.at[src_v], h2_v, sem0)

        @pl.loop(0, CH // 16)
        def _(j):
            sv = src_v[pl.ds(j * 16, 16)]
            dv = dst_v[pl.ds(j * 16, 16)]
            s = plsc.load_gather(as2_v, [sv]) + plsc.load_gather(ad2_v, [dv])
            w_v[pl.ds(j * 16, 16)] = jnp.exp(jnp.maximum(s, 0.2 * s))

        c3.wait()

        @pl.loop(0, CH)
        def _(i):
            sc = plsc.load_gather(w_v, [iota * 0 + i])
            msg_v[i, pl.ds(0, 16)] = h2_v[i] * sc
            msg_v[i, pl.ds(16, 16)] = sc * l0mask

        pltpu.sync_copy(msg_v, acc_sh.at[dst_v], add=True)

    plsc.subcore_barrier()
    ob = cid * NP + row0
    for j in range(RPT // CH):
        pltpu.sync_copy(acc_sh.at[pl.ds(row0 + j * CH, CH)], msg_v)
        pltpu.sync_copy(msg_v, out_hbm.at[pl.ds(ob + j * CH, CH)])


def _sc2(srcE, dstE, as2, ad2, h2):
    k = pl.kernel(
        _sc2_body,
        out_type=jax.ShapeDtypeStruct((NCORE * NP, D2P), jnp.float32),
        mesh=_mesh(),
        compiler_params=_SC_PARAMS,
        scratch_types=[
            pltpu.VMEM((CH,), jnp.int32),
            pltpu.VMEM((CH,), jnp.int32),
            pltpu.VMEM((NP,), jnp.float32),
            pltpu.VMEM((NP,), jnp.float32),
            pltpu.VMEM((CH,), jnp.float32),
            pltpu.VMEM((CH, D2), jnp.float32),
            pltpu.VMEM((CH, D2P), jnp.float32),
            pltpu.VMEM_SHARED((NP, D2P), jnp.float32),
            pltpu.SemaphoreType.DMA,
        ],
    )
    return k(srcE, dstE, as2, ad2, h2)


def _tcc_body(a_ref, b_ref, b2_ref, o_ref):
    acc = a_ref[...] + b_ref[...]
    den = acc[:, D2:D2 + 1]
    o_ref[...] = acc[:, :D2] / (den + 1e-16) + b2_ref[...]


def _tcc(acc2, b2r):
    return pl.pallas_call(
        _tcc_body,
        grid=(NP // BR,),
        in_specs=[
            pl.BlockSpec((BR, D2P), lambda i: (i, 0)),
            pl.BlockSpec((BR, D2P), lambda i: (i + NP // BR, 0)),
            pl.BlockSpec((1, D2), lambda i: (0, 0)),
        ],
        out_specs=pl.BlockSpec((BR, D2), lambda i: (i, 0)),
        out_shape=jax.ShapeDtypeStruct((NP, D2), jnp.float32),
    )(acc2, acc2, b2r)


def kernel(x, edge_index, W1, a_src1, a_dst1, b1, W2, a_src2, a_dst2, b2):
    loop = jnp.arange(N, dtype=jnp.int32)
    padi = jnp.full((EP - E - N,), N, dtype=jnp.int32)
    srcE = jnp.concatenate([edge_index[0], loop, padi])
    dstE = jnp.concatenate([edge_index[1], loop, padi])

    x_pad = jnp.zeros((NP, F_IN), jnp.float32).at[:N].set(x)

    idx = jnp.arange(D1)
    As16 = jnp.zeros((D1, 16), jnp.float32).at[idx, idx // C1].set(
        a_src1.reshape(-1))
    Ad16 = jnp.zeros((D1, 16), jnp.float32).at[idx, idx // C1].set(
        a_dst1.reshape(-1))
    Rexp = jnp.zeros((H, D1), jnp.float32).at[idx // C1, idx].set(1.0)

    b1r = b1.reshape(1, D1)
    b2r = b2.reshape(1, D2)

    ha, hb, ts, td = _tca(
        x_pad, W1[:, :DH], W1[:, DH:], As16[:DH], As16[DH:],
        Ad16[:DH], Ad16[DH:])
    acc1 = _sc1(srcE, dstE, ts, td, ha, hb)
    h2, as2, ad2 = _tcb(acc1, W2[:DH], W2[DH:], b1r, Rexp, a_src2, a_dst2)
    acc2 = _sc2(srcE, dstE, as2, ad2, h2)
    out = _tcc(acc2, b2r)
    return out[:N]


# speedup vs baseline: 45.4624x; 1.3606x over previous
import jax
import jax.numpy as jnp
from jax import lax
from jax.experimental import pallas as pl
from jax.experimental.pallas import tpu as pltpu
from jax.experimental.pallas import tpu_sc as plsc

N = 10000
E = 320000
F_IN = 128
H = 8
C1 = 20
D1 = H * C1
DH = D1 // 2
D2 = 16

NP = 10240
DHP = DH + 16
D2P = 32
NCORE = 2
NSUB = 16
NW = NCORE * NSUB
CH = 128
CPT = 82
PT = CH * CPT
EP = NW * PT
EPX = EP + 2 * CH
RPT = NP // NSUB
BR = 512


def _mesh():
    return plsc.VectorSubcoreMesh(
        core_axis_name="c", subcore_axis_name="s",
        num_cores=NCORE, num_subcores=NSUB)


_SC_PARAMS = pltpu.CompilerParams(
    needs_layout_passes=False, use_tc_tiling_on_sc=False)


def _tca_body(x_ref, wa_ref, wb_ref, asa_ref, asb_ref, ada_ref, adb_ref,
              ha_ref, hb_ref, ts_ref, td_ref):
    x = x_ref[...]
    ha = jnp.dot(x, wa_ref[...], preferred_element_type=jnp.float32)
    hb = jnp.dot(x, wb_ref[...], preferred_element_type=jnp.float32)
    ha_ref[...] = ha
    hb_ref[...] = hb
    ts_ref[...] = (
        jnp.dot(ha, asa_ref[...], preferred_element_type=jnp.float32)
        + jnp.dot(hb, asb_ref[...], preferred_element_type=jnp.float32))
    td_ref[...] = (
        jnp.dot(ha, ada_ref[...], preferred_element_type=jnp.float32)
        + jnp.dot(hb, adb_ref[...], preferred_element_type=jnp.float32))


def _tca(x_pad, W1a, W1b, AsA, AsB, AdA, AdB):
    full = lambda r, c: pl.BlockSpec((r, c), lambda i: (0, 0))
    return pl.pallas_call(
        _tca_body,
        grid=(NP // BR,),
        in_specs=[
            pl.BlockSpec((BR, F_IN), lambda i: (i, 0)),
            full(F_IN, DH), full(F_IN, DH),
            full(DH, 16), full(DH, 16), full(DH, 16), full(DH, 16),
        ],
        out_specs=[
            pl.BlockSpec((BR, DH), lambda i: (i, 0)),
            pl.BlockSpec((BR, DH), lambda i: (i, 0)),
            pl.BlockSpec((BR, 16), lambda i: (i, 0)),
            pl.BlockSpec((BR, 16), lambda i: (i, 0)),
        ],
        out_shape=[
            jax.ShapeDtypeStruct((NP, DH), jnp.float32),
            jax.ShapeDtypeStruct((NP, DH), jnp.float32),
            jax.ShapeDtypeStruct((NP, 16), jnp.float32),
            jax.ShapeDtypeStruct((NP, 16), jnp.float32),
        ],
    )(x_pad, W1a, W1b, AsA, AsB, AdA, AdB)


def _sc1_body(src_hbm, dst_hbm, ts_hbm, td_hbm, ha_hbm, hb_hbm, out_hbm,
              src_a, dst_a, dsts_a, as_a, ad_a, w_a, h_a, msg_a,
              src_b, dst_b, dsts_b, as_b, ad_b, w_b, h_b, msg_b,
              acc_sh, semi_a, semg_a, sems_a, semi_b, semg_b, sems_b):
    cid = lax.axis_index("c")
    sid = lax.axis_index("s")
    wid = sid * NCORE + cid

    iota = lax.iota(jnp.int32, 16)
    hmask = jnp.where(iota < H, 1.0, 0.0).astype(jnp.float32)
    zz = jnp.zeros((16,), jnp.float32)
    row0 = sid * RPT
    ebase0 = wid * PT

    sets = (
        (src_a, dst_a, dsts_a, as_a, ad_a, w_a, h_a, msg_a,
         semi_a, semg_a, sems_a),
        (src_b, dst_b, dsts_b, as_b, ad_b, w_b, h_b, msg_b,
         semi_b, semg_b, sems_b),
    )

    def issue_idx(S, c):
        (src_v, dst_v, _, _, _, _, _, _, semi, _, _) = S
        eb = ebase0 + c * CH
        pltpu.async_copy(src_hbm.at[pl.ds(eb, CH)], src_v, semi)
        pltpu.async_copy(dst_hbm.at[pl.ds(eb, CH)], dst_v, semi)

    def wait_idx(S):
        (src_v, dst_v, _, _, _, _, _, _, semi, _, _) = S
        pltpu.make_async_copy(src_hbm.at[pl.ds(0, CH)], src_v, semi).wait()
        pltpu.make_async_copy(dst_hbm.at[pl.ds(0, CH)], dst_v, semi).wait()

    def issue_gathers(S, h_hbm):
        (src_v, dst_v, _, as_v, ad_v, _, h_v, _, _, semg, _) = S
        pltpu.async_copy(ts_hbm.at[src_v], as_v, semg)
        pltpu.async_copy(td_hbm.at[dst_v], ad_v, semg)
        pltpu.async_copy(h_hbm.at[src_v], h_v, semg)

    def wait_gathers(S, h_hbm):
        (src_v, dst_v, _, as_v, ad_v, _, h_v, _, _, semg, _) = S
        pltpu.make_async_copy(ts_hbm.at[src_v], as_v, semg).wait()
        pltpu.make_async_copy(td_hbm.at[dst_v], ad_v, semg).wait()
        pltpu.make_async_copy(h_hbm.at[src_v], h_v, semg).wait()

    def issue_scatter(S):
        (_, _, dsts_v, _, _, _, _, msg_v, _, _, sems) = S
        pltpu.async_copy(msg_v, acc_sh.at[dsts_v], sems, add=True)

    def wait_scatter(S):
        (_, _, dsts_v, _, _, _, _, msg_v, _, _, sems) = S
        pltpu.make_async_copy(msg_v, acc_sh.at[dsts_v], sems).wait()

    def shadow_dst(S):
        (_, dst_v, dsts_v, _, _, _, _, _, _, _, _) = S

        @pl.loop(0, CH // 16)
        def _(j):
            dsts_v[pl.ds(j * 16, 16)] = dst_v[pl.ds(j * 16, 16)]

    for phase, h_hbm in ((0, ha_hbm), (1, hb_hbm)):
        hmaps = [(iota + 16 * k) // C1 + 4 * phase for k in range(DH // 16)]

        def compute(S, _hmaps=hmaps):
            (_, _, _, as_v, ad_v, w_v, h_v, msg_v, _, _, _) = S

            @pl.loop(0, CH, unroll=2)
            def _(i):
                s = as_v[i] + ad_v[i]
                w = jnp.exp(jnp.maximum(s, 0.2 * s))
                w_v[pl.ds(i * 16, 16)] = w
                msg_v[i, pl.ds(DH, 16)] = w * hmask
                for k in range(DH // 16):
                    sc = plsc.load_gather(w_v, [i * 16 + _hmaps[k]])
                    msg_v[i, pl.ds(k * 16, 16)] = (
                        h_v[i, pl.ds(k * 16, 16)] * sc)

        @pl.loop(0, CH)
        def _(i):
            for k in range(DHP // 16):
                msg_a[i, pl.ds(k * 16, 16)] = zz

        for j in range(RPT // CH):
            pltpu.sync_copy(msg_a, acc_sh.at[pl.ds(row0 + j * CH, CH)])
        plsc.subcore_barrier()

        issue_idx(sets[0], 0)
        issue_idx(sets[1], 1)
        wait_idx(sets[0])
        issue_gathers(sets[0], h_hbm)

        @pl.loop(0, CPT // 2)
        def _(g):
            c0 = g * 2
            A, B = sets

            @pl.when(g > 0)
            def _():
                wait_scatter(A)

            wait_gathers(A, h_hbm)
            shadow_dst(A)
            issue_idx(A, c0 + 2)
            wait_idx(B)
            issue_gathers(B, h_hbm)
            compute(A)
            issue_scatter(A)

            @pl.when(g > 0)
            def _():
                wait_scatter(B)

            wait_gathers(B, h_hbm)
            shadow_dst(B)
            issue_idx(B, c0 + 3)
            wait_idx(A)
            issue_gathers(A, h_hbm)
            compute(B)
            issue_scatter(B)

        wait_gathers(sets[0], h_hbm)
        wait_idx(sets[1])
        wait_scatter(sets[0])
        wait_scatter(sets[1])

        plsc.subcore_barrier()
        ob = (cid * 2 + phase) * NP + row0
        for j in range(RPT // CH):
            pltpu.sync_copy(acc_sh.at[pl.ds(row0 + j * CH, CH)], msg_a)
            pltpu.sync_copy(msg_a, out_hbm.at[pl.ds(ob + j * CH, CH)])
        plsc.subcore_barrier()


def _sc1(srcE, dstE, ts, td, ha, hb):
    buf = lambda: [
        pltpu.VMEM((CH,), jnp.int32),
        pltpu.VMEM((CH,), jnp.int32),
        pltpu.VMEM((CH,), jnp.int32),
        pltpu.VMEM((CH, 16), jnp.float32),
        pltpu.VMEM((CH, 16), jnp.float32),
        pltpu.VMEM((CH * 16,), jnp.float32),
        pltpu.VMEM((CH, DH), jnp.float32),
        pltpu.VMEM((CH, DHP), jnp.float32),
    ]
    k = pl.kernel(
        _sc1_body,
        out_type=jax.ShapeDtypeStruct((NCORE * 2 * NP, DHP), jnp.float32),
        mesh=_mesh(),
        compiler_params=_SC_PARAMS,
        scratch_types=buf() + buf() + [
            pltpu.VMEM_SHARED((NP, DHP), jnp.float32),
            pltpu.SemaphoreType.DMA,
            pltpu.SemaphoreType.DMA,
            pltpu.SemaphoreType.DMA,
            pltpu.SemaphoreType.DMA,
            pltpu.SemaphoreType.DMA,
            pltpu.SemaphoreType.DMA,
        ],
    )
    return k(srcE, dstE, ts, td, ha, hb)


def _tcb_body(p0a_ref, p0b_ref, p1a_ref, p1b_ref, w2a_ref, w2b_ref,
              b1_ref, rexp_ref, as2_ref, ad2_ref,
              h2_ref, s2_ref, d2_ref):
    accA = p0a_ref[...] + p0b_ref[...]
    accB = p1a_ref[...] + p1b_ref[...]
    den = accA[:, DH:DH + H]
    rec = 1.0 / (den + 1e-16)
    rec160 = jnp.dot(rec, rexp_ref[...], preferred_element_type=jnp.float32)
    outA = jnp.maximum(
        accA[:, :DH] * rec160[:, :DH] + b1_ref[:, :DH], 0.0)
    outB = jnp.maximum(
        accB[:, :DH] * rec160[:, DH:] + b1_ref[:, DH:], 0.0)
    h2 = (jnp.dot(outA, w2a_ref[...], preferred_element_type=jnp.float32)
          + jnp.dot(outB, w2b_ref[...], preferred_element_type=jnp.float32))
    h2_ref[...] = h2
    s2_ref[...] = jnp.sum(h2 * as2_ref[...], axis=1)
    d2_ref[...] = jnp.sum(h2 * ad2_ref[...], axis=1)


def _tcb(acc1, W2a, W2b, b1r, Rexp, as2r, ad2r):
    full = lambda r, c: pl.BlockSpec((r, c), lambda i: (0, 0))
    nb = NP // BR
    return pl.pallas_call(
        _tcb_body,
        grid=(nb,),
        in_specs=[
            pl.BlockSpec((BR, DHP), lambda i: (i, 0)),
            pl.BlockSpec((BR, DHP), lambda i: (i + 2 * nb, 0)),
            pl.BlockSpec((BR, DHP), lambda i: (i + nb, 0)),
            pl.BlockSpec((BR, DHP), lambda i: (i + 3 * nb, 0)),
            full(DH, D2), full(DH, D2),
            full(1, D1), full(H, D1), full(1, D2), full(1, D2),
        ],
        out_specs=[
            pl.BlockSpec((BR, D2), lambda i: (i, 0)),
            pl.BlockSpec((BR,), lambda i: (i,)),
            pl.BlockSpec((BR,), lambda i: (i,)),
        ],
        out_shape=[
            jax.ShapeDtypeStruct((NP, D2), jnp.float32),
            jax.ShapeDtypeStruct((NP,), jnp.float32),
            jax.ShapeDtypeStruct((NP,), jnp.float32),
        ],
    )(acc1, acc1, acc1, acc1, W2a, W2b, b1r, Rexp, as2r, ad2r)


def _sc2_body(src_hbm, dst_hbm, as2_hbm, ad2_hbm, h2_hbm, out_hbm,
              src_a, dst_a, dsts_a, w_a, h2_a, msg_a,
              src_b, dst_b, dsts_b, w_b, h2_b, msg_b,
              as2_v, ad2_v, acc_sh,
              semi_a, semg_a, sems_a, semi_b, semg_b, sems_b):
    cid = lax.axis_index("c")
    sid = lax.axis_index("s")
    wid = sid * NCORE + cid

    iota = lax.iota(jnp.int32, 16)
    l0mask = jnp.where(iota == 0, 1.0, 0.0).astype(jnp.float32)
    zz = jnp.zeros((16,), jnp.float32)
    row0 = sid * RPT
    ebase0 = wid * PT

    pltpu.sync_copy(as2_hbm, as2_v)
    pltpu.sync_copy(ad2_hbm, ad2_v)

    sets = (
        (src_a, dst_a, dsts_a, w_a, h2_a, msg_a, semi_a, semg_a, sems_a),
        (src_b, dst_b, dsts_b, w_b, h2_b, msg_b, semi_b, semg_b, sems_b),
    )

    def issue_idx(S, c):
        (src_v, dst_v, _, _, _, _, semi, _, _) = S
        eb = ebase0 + c * CH
        pltpu.async_copy(src_hbm.at[pl.ds(eb, CH)], src_v, semi)
        pltpu.async_copy(dst_hbm.at[pl.ds(eb, CH)], dst_v, semi)

    def wait_idx(S):
        (src_v, dst_v, _, _, _, _, semi, _, _) = S
        pltpu.make_async_copy(src_hbm.at[pl.ds(0, CH)], src_v, semi).wait()
        pltpu.make_async_copy(dst_hbm.at[pl.ds(0, CH)], dst_v, semi).wait()

    def issue_gathers(S):
        (src_v, _, _, _, h2_v, _, _, semg, _) = S
        pltpu.async_copy(h2_hbm.at[src_v], h2_v, semg)

    def wait_gathers(S):
        (src_v, _, _, _, h2_v, _, _, semg, _) = S
        pltpu.make_async_copy(h2_hbm.at[src_v], h2_v, semg).wait()

    def issue_scatter(S):
        (_, _, dsts_v, _, _, msg_v, _, _, sems) = S
        pltpu.async_copy(msg_v, acc_sh.at[dsts_v], sems, add=True)

    def wait_scatter(S):
        (_, _, dsts_v, _, _, msg_v, _, _, sems) = S
        pltpu.make_async_copy(msg_v, acc_sh.at[dsts_v], sems).wait()

    def shadow_dst(S):
        (_, dst_v, dsts_v, _, _, _, _, _, _) = S

        @pl.loop(0, CH // 16)
        def _(j):
            dsts_v[pl.ds(j * 16, 16)] = dst_v[pl.ds(j * 16, 16)]

    def compute(S):
        (src_v, dst_v, _, w_v, h2_v, msg_v, _, _, _) = S

        @pl.loop(0, CH // 16)
        def _(j):
            sv = src_v[pl.ds(j * 16, 16)]
            dv = dst_v[pl.ds(j * 16, 16)]
            s = plsc.load_gather(as2_v, [sv]) + plsc.load_gather(ad2_v, [dv])
            w_v[pl.ds(j * 16, 16)] = jnp.exp(jnp.maximum(s, 0.2 * s))

        @pl.loop(0, CH, unroll=2)
        def _(i):
            sc = plsc.load_gather(w_v, [iota * 0 + i])
            msg_v[i, pl.ds(0, 16)] = h2_v[i] * sc
            msg_v[i, pl.ds(16, 16)] = sc * l0mask

    @pl.loop(0, CH)
    def _(i):
        msg_a[i, pl.ds(0, 16)] = zz
        msg_a[i, pl.ds(16, 16)] = zz

    for j in range(RPT // CH):
        pltpu.sync_copy(msg_a, acc_sh.at[pl.ds(row0 + j * CH, CH)])
    plsc.subcore_barrier()

    issue_idx(sets[0], 0)
    issue_idx(sets[1], 1)
    wait_idx(sets[0])
    issue_gathers(sets[0])

    @pl.loop(0, CPT // 2)
    def _(g):
        c0 = g * 2
        A, B = sets

        @pl.when(g > 0)
        def _():
            wait_scatter(A)

        wait_gathers(A)
        shadow_dst(A)
        issue_idx(A, c0 + 2)
        wait_idx(B)
        issue_gathers(B)
        compute(A)
        issue_scatter(A)

        @pl.when(g > 0)
        def _():
            wait_scatter(B)

        wait_gathers(B)
        shadow_dst(B)
        issue_idx(B, c0 + 3)
        wait_idx(A)
        issue_gathers(A)
        compute(B)
        issue_scatter(B)

    wait_gathers(sets[0])
    wait_idx(sets[1])
    wait_scatter(sets[0])
    wait_scatter(sets[1])

    plsc.subcore_barrier()
    ob = cid * NP + row0
    for j in range(RPT // CH):
        pltpu.sync_copy(acc_sh.at[pl.ds(row0 + j * CH, CH)], msg_a)
        pltpu.sync_copy(msg_a, out_hbm.at[pl.ds(ob + j * CH, CH)])


def _sc2(srcE, dstE, as2, ad2, h2):
    buf = lambda: [
        pltpu.VMEM((CH,), jnp.int32),
        pltpu.VMEM((CH,), jnp.int32),
        pltpu.VMEM((CH,), jnp.int32),
        pltpu.VMEM((CH,), jnp.float32),
        pltpu.VMEM((CH, D2), jnp.float32),
        pltpu.VMEM((CH, D2P), jnp.float32),
    ]
    k = pl.kernel(
        _sc2_body,
        out_type=jax.ShapeDtypeStruct((NCORE * NP, D2P), jnp.float32),
        mesh=_mesh(),
        compiler_params=_SC_PARAMS,
        scratch_types=buf() + buf() + [
            pltpu.VMEM((NP,), jnp.float32),
            pltpu.VMEM((NP,), jnp.float32),
            pltpu.VMEM_SHARED((NP, D2P), jnp.float32),
            pltpu.SemaphoreType.DMA,
            pltpu.SemaphoreType.DMA,
            pltpu.SemaphoreType.DMA,
            pltpu.SemaphoreType.DMA,
            pltpu.SemaphoreType.DMA,
            pltpu.SemaphoreType.DMA,
        ],
    )
    return k(srcE, dstE, as2, ad2, h2)


def _tcc_body(a_ref, b_ref, b2_ref, o_ref):
    acc = a_ref[...] + b_ref[...]
    den = acc[:, D2:D2 + 1]
    o_ref[...] = acc[:, :D2] / (den + 1e-16) + b2_ref[...]


def _tcc(acc2, b2r):
    return pl.pallas_call(
        _tcc_body,
        grid=(NP // BR,),
        in_specs=[
            pl.BlockSpec((BR, D2P), lambda i: (i, 0)),
            pl.BlockSpec((BR, D2P), lambda i: (i + NP // BR, 0)),
            pl.BlockSpec((1, D2), lambda i: (0, 0)),
        ],
        out_specs=pl.BlockSpec((BR, D2), lambda i: (i, 0)),
        out_shape=jax.ShapeDtypeStruct((NP, D2), jnp.float32),
    )(acc2, acc2, b2r)


def kernel(x, edge_index, W1, a_src1, a_dst1, b1, W2, a_src2, a_dst2, b2):
    loop = jnp.arange(N, dtype=jnp.int32)
    padi = jnp.full((EPX - E - N,), N, dtype=jnp.int32)
    srcE = jnp.concatenate([edge_index[0], loop, padi])
    dstE = jnp.concatenate([edge_index[1], loop, padi])

    x_pad = jnp.zeros((NP, F_IN), jnp.float32).at[:N].set(x)

    idx = jnp.arange(D1)
    As16 = jnp.zeros((D1, 16), jnp.float32).at[idx, idx // C1].set(
        a_src1.reshape(-1))
    Ad16 = jnp.zeros((D1, 16), jnp.float32).at[idx, idx // C1].set(
        a_dst1.reshape(-1))
    Rexp = jnp.zeros((H, D1), jnp.float32).at[idx // C1, idx].set(1.0)

    b1r = b1.reshape(1, D1)
    b2r = b2.reshape(1, D2)

    ha, hb, ts, td = _tca(
        x_pad, W1[:, :DH], W1[:, DH:], As16[:DH], As16[DH:],
        Ad16[:DH], Ad16[DH:])
    acc1 = _sc1(srcE, dstE, ts, td, ha, hb)
    h2, as2, ad2 = _tcb(acc1, W2[:DH], W2[DH:], b1r, Rexp, a_src2, a_dst2)
    acc2 = _sc2(srcE, dstE, as2, ad2, h2)
    out = _tcc(acc2, b2r)
    return out[:N]
